# Initial kernel scaffold; baseline (speedup 1.0000x reference)
import jax
import jax.numpy as jnp
from jax.experimental import pallas as pl


def kernel(seq, target, table, W1, b1, W2, b2, Wr, br):
    raise NotImplementedError("write your pallas kernel here")


# speedup vs baseline: 12.3946x; 12.3946x over previous
import functools

import jax
import jax.numpy as jnp
from jax import lax
from jax.experimental import pallas as pl
from jax.experimental.pallas import tpu as pltpu
from jax.experimental.pallas import tpu_sc as plsc


def _fold_tables(tableT, W1, b1, Wr, br):
    D, V = tableT.shape
    H1 = W1.shape[1]
    H2 = Wr.shape[1]
    BLK = 2048
    nblk = pl.cdiv(V, BLK)
    Vp = nblk * BLK

    wpad = jnp.zeros((D, 128), jnp.float32)
    wpad = wpad.at[:, 0:H1].set(W1).at[:, H1:H1 + H2].set(Wr)
    bpad = jnp.zeros((128,), jnp.float32)
    bpad = bpad.at[0:H1].set(b1).at[H1:H1 + H2].set(br)

    def body(t_ref, w_ref, b_ref, o_ref):
        x = lax.dot_general(
            t_ref[...], w_ref[...], (((0,), (0,)), ((), ())),
            preferred_element_type=jnp.float32)
        g = jax.nn.sigmoid(x + b_ref[...][None, :])
        row = pl.program_id(0) * BLK + lax.broadcasted_iota(jnp.int32, g.shape, 0)
        lane = lax.broadcasted_iota(jnp.int32, g.shape, 1)
        o_ref[...] = jnp.where((row == 0) & (lane < H1), 0.0, g)

    packed = pl.pallas_call(
        body,
        grid=(nblk,),
        in_specs=[
            pl.BlockSpec((D, BLK), lambda i: (0, i)),
            pl.BlockSpec((D, 128), lambda i: (0, 0)),
            pl.BlockSpec((128,), lambda i: (0,)),
        ],
        out_specs=pl.BlockSpec((BLK, 128), lambda i: (i, 0)),
        out_shape=jax.ShapeDtypeStruct((Vp, 128), jnp.float32),
    )(tableT, wpad, bpad)
    return packed.reshape(8 * Vp, 16)


def _sc_pool_and_target(seq8, tgt81, gview):
    B, L = seq8.shape
    info = plsc.get_sparse_core_info()
    NC, NS = info.num_cores, info.num_subcores
    NW = NC * NS
    PB = B // NW

    nfull = L // 16
    rem = L % 16
    L4 = (L // 4) * 4
    n1 = min(L, 128)

    mesh = plsc.VectorSubcoreMesh(core_axis_name="c", subcore_axis_name="s")

    @functools.partial(
        pl.kernel,
        out_type=[
            jax.ShapeDtypeStruct((B, 32), jnp.float32),
            jax.ShapeDtypeStruct((B, 16), jnp.float32),
        ],
        mesh=mesh,
        compiler_params=pltpu.CompilerParams(use_tc_tiling_on_sc=False),
        scratch_types=[
            pltpu.VMEM((PB, L), jnp.int32),
            pltpu.VMEM((L, 16), jnp.float32),
            pltpu.VMEM((PB, 32), jnp.float32),
            pltpu.VMEM((PB,), jnp.int32),
            pltpu.VMEM((PB, 16), jnp.float32),
            pltpu.SemaphoreType.DMA,
            pltpu.SemaphoreType.DMA,
        ],
    )
    def k(seq_h, tgt_h, g_h, psc_h, th_h,
          seq_v, buf_v, psc_v, tgt_v, th_v, sem, sem2):
        wid = lax.axis_index("s") * NC + lax.axis_index("c")
        base = wid * PB
        pltpu.sync_copy(seq_h.at[pl.ds(base, PB)], seq_v)
        pltpu.sync_copy(tgt_h.at[pl.ds(base, PB)], tgt_v)
        tcopy = pltpu.async_copy(g_h.at[tgt_v], th_v, sem2)

        def body(b, carry):
            c1 = pltpu.async_copy(
                g_h.at[seq_v.at[b, pl.ds(0, n1)]], buf_v.at[pl.ds(0, n1)], sem)
            if L > 128:
                c2 = pltpu.async_copy(
                    g_h.at[seq_v.at[b, pl.ds(128, L - 128)]],
                    buf_v.at[pl.ds(128, L - 128)], sem)
            c1.wait()
            if L > 128:
                c2.wait()

            zero = jnp.zeros((16,), jnp.float32)

            def acc_body(j, accs):
                a0, a1, a2, a3 = accs
                return (a0 + buf_v[4 * j, :], a1 + buf_v[4 * j + 1, :],
                        a2 + buf_v[4 * j + 2, :], a3 + buf_v[4 * j + 3, :])

            a0, a1, a2, a3 = lax.fori_loop(
                0, L4 // 4, acc_body, (zero, zero, zero, zero))
            acc = (a0 + a1) + (a2 + a3)
            for j in range(L4, L):
                acc = acc + buf_v[j, :]

            cntv = jnp.zeros((16,), jnp.int32)
            for kk in range(nfull):
                v = seq_v[b, pl.ds(kk * 16, 16)]
                cntv = cntv + jnp.where(v != 0, 1, 0)
            if rem:
                v = seq_v[b, pl.ds(L - 16, 16)]
                lane = lax.iota(jnp.int32, 16)
                cntv = cntv + jnp.where((v != 0) & (lane >= (16 - rem)), 1, 0)

            psc_v[b, pl.ds(0, 16)] = acc
            psc_v[b, pl.ds(16, 16)] = cntv.astype(jnp.float32)
            return carry

        lax.fori_loop(0, PB, body, 0)
        tcopy.wait()
        pltpu.sync_copy(psc_v, psc_h.at[pl.ds(base, PB)])
        pltpu.sync_copy(th_v, th_h.at[pl.ds(base, PB)])

    return k(seq8, tgt81, gview)


def _tail(psc, th, W2, b2):
    B = psc.shape[0]
    H1 = W2.shape[0]
    H2 = W2.shape[1]

    def body(ps_ref, th_ref, w2_ref, b2_ref, o_ref):
        ps = ps_ref[...]
        acc = ps[:, 0:H1]
        cnt = jnp.sum(ps[:, H1:2 * H1], axis=1, keepdims=True)
        pooled = acc / jnp.maximum(cnt, 1.0)
        sh = jax.nn.sigmoid(
            jnp.dot(pooled, w2_ref[...], preferred_element_type=jnp.float32)
            + b2_ref[...][None, :])
        o_ref[...] = jnp.sum(sh * th_ref[...][:, 0:H2], axis=1, keepdims=True)

    return pl.pallas_call(
        body,
        in_specs=[
            pl.BlockSpec((B, 2 * H1), lambda: (0, 0)),
            pl.BlockSpec((B, 16), lambda: (0, 0)),
            pl.BlockSpec((H1, H2), lambda: (0, 0)),
            pl.BlockSpec((H2,), lambda: (0,)),
        ],
        out_specs=pl.BlockSpec((B, 1), lambda: (0, 0)),
        out_shape=jax.ShapeDtypeStruct((B, 1), jnp.float32),
    )(psc, th, W2, b2)


def kernel(seq, target, table, W1, b1, W2, b2, Wr, br):
    B, L = seq.shape
    gview = _fold_tables(table.T, W1, b1, Wr, br)
    seq8 = seq * 8
    tgt81 = target.reshape(B) * 8 + 1
    psc, th = _sc_pool_and_target(seq8, tgt81, gview)
    out = _tail(psc, th, W2, b2)
    return out.reshape(B, target.shape[1] * target.shape[2])


# speedup vs baseline: 16.6990x; 1.3473x over previous
import functools

import jax
import jax.numpy as jnp
from jax import lax
from jax.experimental import pallas as pl
from jax.experimental.pallas import tpu as pltpu
from jax.experimental.pallas import tpu_sc as plsc


def _fold_tables(tableT, W1, b1, Wr, br):
    D, V = tableT.shape
    H1 = W1.shape[1]
    H2 = Wr.shape[1]
    BLK = 2048
    nblk = pl.cdiv(V, BLK)
    Vp = nblk * BLK

    wpad = jnp.zeros((D, 128), jnp.float32)
    wpad = wpad.at[:, 0:H1].set(W1).at[:, H1:H1 + H2].set(Wr)
    bpad = jnp.zeros((128,), jnp.float32)
    bpad = bpad.at[0:H1].set(b1).at[H1:H1 + H2].set(br)

    def body(t_ref, w_ref, b_ref, o_ref):
        x = lax.dot_general(
            t_ref[...], w_ref[...], (((0,), (0,)), ((), ())),
            preferred_element_type=jnp.float32)
        g = jax.nn.sigmoid(x + b_ref[...][None, :])
        row = pl.program_id(0) * BLK + lax.broadcasted_iota(jnp.int32, g.shape, 0)
        lane = lax.broadcasted_iota(jnp.int32, g.shape, 1)
        o_ref[...] = jnp.where((row == 0) & (lane < H1), 0.0, g)

    packed = pl.pallas_call(
        body,
        grid=(nblk,),
        in_specs=[
            pl.BlockSpec((D, BLK), lambda i: (0, i)),
            pl.BlockSpec((D, 128), lambda i: (0, 0)),
            pl.BlockSpec((128,), lambda i: (0,)),
        ],
        out_specs=pl.BlockSpec((BLK, 128), lambda i: (i, 0)),
        out_shape=jax.ShapeDtypeStruct((Vp, 128), jnp.float32),
    )(tableT, wpad, bpad)
    return packed.reshape(8 * Vp, 16)


def _sc_pool_and_target(seq8, tgt81, gview):
    B, L = seq8.shape
    info = plsc.get_sparse_core_info()
    NC, NS = info.num_cores, info.num_subcores
    NW = NC * NS
    PB = B // NW

    nfull = L // 16
    rem = L % 16
    n1 = min(L, 128)

    mesh = plsc.VectorSubcoreMesh(core_axis_name="c", subcore_axis_name="s")

    @functools.partial(
        pl.kernel,
        out_type=[
            jax.ShapeDtypeStruct((B, 32), jnp.float32),
            jax.ShapeDtypeStruct((B, 16), jnp.float32),
        ],
        mesh=mesh,
        compiler_params=pltpu.CompilerParams(use_tc_tiling_on_sc=False),
        scratch_types=[
            pltpu.VMEM((PB, L), jnp.int32),
            pltpu.VMEM((2, L, 16), jnp.float32),
            pltpu.VMEM((PB, 32), jnp.float32),
            pltpu.VMEM((PB,), jnp.int32),
            pltpu.VMEM((PB, 16), jnp.float32),
            pltpu.SemaphoreType.DMA,
            pltpu.SemaphoreType.DMA,
            pltpu.SemaphoreType.DMA,
        ],
    )
    def k(seq_h, tgt_h, g_h, psc_h, th_h,
          seq_v, buf_v, psc_v, tgt_v, th_v, sem_a, sem_b, sem_t):
        wid = lax.axis_index("s") * NC + lax.axis_index("c")
        base = wid * PB
        pltpu.sync_copy(seq_h.at[pl.ds(base, PB)], seq_v)
        pltpu.sync_copy(tgt_h.at[pl.ds(base, PB)], tgt_v)
        tcopy = pltpu.async_copy(g_h.at[tgt_v], th_v, sem_t)

        sems = (sem_a, sem_b)

        def issue(b, slot):
            sem = sems[slot]
            pltpu.async_copy(
                g_h.at[seq_v.at[b, pl.ds(0, n1)]],
                buf_v.at[slot, pl.ds(0, n1)], sem)
            if L > 128:
                pltpu.async_copy(
                    g_h.at[seq_v.at[b, pl.ds(128, L - 128)]],
                    buf_v.at[slot, pl.ds(128, L - 128)], sem)

        def drain(slot):
            sem = sems[slot]
            pltpu.make_async_copy(
                g_h.at[pl.ds(0, n1)], buf_v.at[slot, pl.ds(0, n1)],
                sem).wait()
            if L > 128:
                pltpu.make_async_copy(
                    g_h.at[pl.ds(0, L - 128)],
                    buf_v.at[slot, pl.ds(128, L - 128)], sem).wait()

        def process(b, slot):
            cntv = jnp.zeros((16,), jnp.int32)
            for kk in range(nfull):
                v = seq_v[b, pl.ds(kk * 16, 16)]
                cntv = cntv + jnp.where(v != 0, 1, 0)
            if rem:
                v = seq_v[b, pl.ds(L - 16, 16)]
                lane = lax.iota(jnp.int32, 16)
                cntv = cntv + jnp.where((v != 0) & (lane >= (16 - rem)), 1, 0)

            drain(slot)
            zero = jnp.zeros((16,), jnp.float32)
            accs = [zero, zero, zero, zero]
            for j in range(L):
                accs[j % 4] = accs[j % 4] + buf_v[slot, j, :]
            acc = (accs[0] + accs[1]) + (accs[2] + accs[3])

            psc_v[b, pl.ds(0, 16)] = acc
            psc_v[b, pl.ds(16, 16)] = cntv.astype(jnp.float32)

        issue(0, 0)
        issue(1, 1)

        def body(i, carry):
            g = 2 * i
            process(g, 0)
            issue(jnp.minimum(g + 2, PB - 1), 0)
            process(g + 1, 1)
            issue(jnp.minimum(g + 3, PB - 1), 1)
            return carry

        lax.fori_loop(0, PB // 2, body, 0)
        drain(0)
        drain(1)
        tcopy.wait()
        pltpu.sync_copy(psc_v, psc_h.at[pl.ds(base, PB)])
        pltpu.sync_copy(th_v, th_h.at[pl.ds(base, PB)])

    return k(seq8, tgt81, gview)


def _tail(psc, th, W2, b2):
    B = psc.shape[0]
    H1 = W2.shape[0]
    H2 = W2.shape[1]

    def body(ps_ref, th_ref, w2_ref, b2_ref, o_ref):
        ps = ps_ref[...]
        acc = ps[:, 0:H1]
        cnt = jnp.sum(ps[:, H1:2 * H1], axis=1, keepdims=True)
        pooled = acc / jnp.maximum(cnt, 1.0)
        sh = jax.nn.sigmoid(
            jnp.dot(pooled, w2_ref[...], preferred_element_type=jnp.float32)
            + b2_ref[...][None, :])
        o_ref[...] = jnp.sum(sh * th_ref[...][:, 0:H2], axis=1, keepdims=True)

    return pl.pallas_call(
        body,
        in_specs=[
            pl.BlockSpec((B, 2 * H1), lambda: (0, 0)),
            pl.BlockSpec((B, 16), lambda: (0, 0)),
            pl.BlockSpec((H1, H2), lambda: (0, 0)),
            pl.BlockSpec((H2,), lambda: (0,)),
        ],
        out_specs=pl.BlockSpec((B, 1), lambda: (0, 0)),
        out_shape=jax.ShapeDtypeStruct((B, 1), jnp.float32),
    )(psc, th, W2, b2)


def kernel(seq, target, table, W1, b1, W2, b2, Wr, br):
    B, L = seq.shape
    gview = _fold_tables(table.T, W1, b1, Wr, br)
    seq8 = seq * 8
    tgt81 = target.reshape(B) * 8 + 1
    psc, th = _sc_pool_and_target(seq8, tgt81, gview)
    out = _tail(psc, th, W2, b2)
    return out.reshape(B, target.shape[1] * target.shape[2])
